# baseline (speedup 1.0000x reference)
import functools

import jax
import jax.numpy as jnp
from jax import lax
from jax.experimental import pallas as pl
from jax.experimental.pallas import tpu as pltpu
from jax.experimental.pallas import tpu_sc as plsc

_B, _S, _D = 16, 2048, 32
_K = 8192
_KMIX = 8
_T = _B * _S
_NC, _NS = 2, 16
_NW = _NC * _NS
_TPW = _T // _NW
_CHUNK = 128

_TT = 1024
_KT = 512


def _sc_mesh():
    return plsc.VectorSubcoreMesh(
        core_axis_name="c", subcore_axis_name="s",
        num_cores=_NC, num_subcores=_NS)


def _embed_gather(ids, emb):
    nchunk = _TPW // _CHUNK

    @functools.partial(
        pl.kernel,
        out_type=jax.ShapeDtypeStruct((_T, _D), jnp.float32),
        mesh=_sc_mesh(),
        compiler_params=pltpu.CompilerParams(use_tc_tiling_on_sc=False),
        scratch_types=[
            pltpu.VMEM((_CHUNK,), jnp.int32),
            pltpu.VMEM((_CHUNK, _D), jnp.float32),
            pltpu.SemaphoreType.DMA,
        ],
    )
    def k(ids_hbm, emb_hbm, out_hbm, idx_v, rows_v, sem):
        wid = lax.axis_index("s") * _NC + lax.axis_index("c")
        t0 = wid * _TPW

        def chunk(c, carry):
            off = t0 + c * _CHUNK
            pltpu.sync_copy(ids_hbm.at[pl.ds(off, _CHUNK)], idx_v)
            pltpu.async_copy(emb_hbm.at[idx_v], rows_v, sem).wait()
            pltpu.sync_copy(rows_v, out_hbm.at[pl.ds(off, _CHUNK)])
            return carry

        lax.fori_loop(0, nchunk, chunk, 0)

    return k(ids, emb)


_DA = 40


def _tc_argmin(base2d, maskf, cbta):
    nk = _K // _KT

    def body(base_ref, mask_ref, cbta_ref, out_ref):
        base = base_ref[...] * mask_ref[...]
        ones = jnp.ones((_TT, 3), jnp.float32)
        zeros = jnp.zeros((_TT, _DA - _D - 3), jnp.float32)
        ba = jnp.concatenate([base, ones, zeros], axis=1)
        best = None
        bidx = None
        for kk in range(nk):
            cb = cbta_ref[:, kk * _KT:(kk + 1) * _KT]
            d = jnp.dot(ba, cb, preferred_element_type=jnp.float32)
            m = jnp.min(d, axis=1, keepdims=True)
            loc = jnp.argmin(d, axis=1).astype(jnp.int32)
            idx = loc[:, None] + kk * _KT
            if kk == 0:
                best, bidx = m, idx
            else:
                upd = m < best
                bidx = jnp.where(upd, idx, bidx)
                best = jnp.where(upd, m, best)
        out_ref[...] = bidx

    return pl.pallas_call(
        body,
        grid=(_T // _TT,),
        in_specs=[
            pl.BlockSpec((_TT, _D), lambda t: (t, 0)),
            pl.BlockSpec((_TT, 1), lambda t: (t, 0)),
            pl.BlockSpec((_DA, _K), lambda t: (0, 0)),
        ],
        out_specs=pl.BlockSpec((_TT, 1), lambda t: (t, 0)),
        out_shape=jax.ShapeDtypeStruct((_T, 1), jnp.int32),
    )(base2d, maskf, cbta)


def _mix_gather(qidx, off, codebook):
    tok_per_chunk = _CHUNK // _KMIX
    nchunk = _TPW // tok_per_chunk
    opw = _TPW * _KMIX

    @functools.partial(
        pl.kernel,
        out_type=jax.ShapeDtypeStruct((_T, _D), jnp.float32),
        mesh=_sc_mesh(),
        compiler_params=pltpu.CompilerParams(
            use_tc_tiling_on_sc=False, needs_layout_passes=False),
        scratch_types=[
            pltpu.VMEM((_T,), jnp.int32),
            pltpu.VMEM((opw,), jnp.int32),
            pltpu.VMEM((_CHUNK,), jnp.int32),
            pltpu.VMEM((_CHUNK, _D), jnp.float32),
            pltpu.VMEM((_TPW, _D), jnp.float32),
            pltpu.SemaphoreType.DMA,
        ],
    )
    def k(qidx_hbm, off_hbm, cb_hbm, out_hbm,
          qidx_v, off_v, codes_v, rows_v, out_v, sem):
        wid = lax.axis_index("s") * _NC + lax.axis_index("c")
        pltpu.sync_copy(qidx_hbm, qidx_v)
        pltpu.sync_copy(off_hbm.at[pl.ds(wid * opw, opw)], off_v)

        def chunk(c, carry):
            cbase = c * _CHUNK
            for g in range(_CHUNK // 16):
                off16 = off_v[pl.ds(cbase + g * 16, 16)]
                codes_v[pl.ds(g * 16, 16)] = plsc.load_gather(qidx_v, [off16])
            pltpu.async_copy(cb_hbm.at[codes_v], rows_v, sem).wait()
            for i in range(tok_per_chunk):
                for h in range(_D // 16):
                    acc = rows_v[i * _KMIX, pl.ds(h * 16, 16)]
                    for j in range(1, _KMIX):
                        acc = acc + rows_v[i * _KMIX + j, pl.ds(h * 16, 16)]
                    out_v[c * tok_per_chunk + i, pl.ds(h * 16, 16)] = (
                        acc * (1.0 / _KMIX))
            return carry

        lax.fori_loop(0, nchunk, chunk, 0)
        pltpu.sync_copy(out_v, out_hbm.at[pl.ds(wid * _TPW, _TPW)])

    return k(qidx, off, codebook)


def kernel(input_ids, attention_mask, token_embedding, codebook):
    ids = input_ids.reshape(-1)
    maskf = attention_mask.reshape(-1, 1).astype(jnp.float32)
    base = _embed_gather(ids, token_embedding)
    cbt2 = -2.0 * codebook.T
    c2 = jnp.sum(codebook * codebook, axis=1)
    c2h = c2.astype(jnp.bfloat16).astype(jnp.float32)
    r = c2 - c2h
    c2m = r.astype(jnp.bfloat16).astype(jnp.float32)
    c2l = r - c2m
    cbta = jnp.concatenate(
        [cbt2, c2h[None], c2m[None], c2l[None],
         jnp.zeros((_DA - _D - 3, _K), jnp.float32)], axis=0)
    qidx = _tc_argmin(base, maskf, cbta).reshape(-1)
    mix = jax.random.randint(jax.random.key(1), (_B, _S, _KMIX), 0, _B)
    off = (mix * _S
           + jnp.arange(_S, dtype=jnp.int32)[None, :, None]).reshape(-1)
    mixed = _mix_gather(qidx, off.astype(jnp.int32), codebook)
    return mixed.reshape(_B, _S, _D)


# speedup vs baseline: 1.6184x; 1.6184x over previous
import functools

import jax
import jax.numpy as jnp
from jax import lax
from jax.experimental import pallas as pl
from jax.experimental.pallas import tpu as pltpu
from jax.experimental.pallas import tpu_sc as plsc

_B, _S, _D = 16, 2048, 32
_K = 8192
_KMIX = 8
_T = _B * _S
_NC, _NS = 2, 16
_NW = _NC * _NS
_TPW = _T // _NW
_CHUNK = 128

_TT = 1024
_KT = 512


def _sc_mesh():
    return plsc.VectorSubcoreMesh(
        core_axis_name="c", subcore_axis_name="s",
        num_cores=_NC, num_subcores=_NS)


def _embed_gather(ids, emb):
    nchunk = _TPW // _CHUNK

    @functools.partial(
        pl.kernel,
        out_type=jax.ShapeDtypeStruct((_T, _D), jnp.float32),
        mesh=_sc_mesh(),
        compiler_params=pltpu.CompilerParams(use_tc_tiling_on_sc=False),
        scratch_types=[
            pltpu.VMEM((_CHUNK,), jnp.int32),
            pltpu.VMEM((_CHUNK, _D), jnp.float32),
            pltpu.SemaphoreType.DMA,
        ],
    )
    def k(ids_hbm, emb_hbm, out_hbm, idx_v, rows_v, sem):
        wid = lax.axis_index("s") * _NC + lax.axis_index("c")
        t0 = wid * _TPW

        def chunk(c, carry):
            off = t0 + c * _CHUNK
            pltpu.sync_copy(ids_hbm.at[pl.ds(off, _CHUNK)], idx_v)
            pltpu.async_copy(emb_hbm.at[idx_v], rows_v, sem).wait()
            pltpu.sync_copy(rows_v, out_hbm.at[pl.ds(off, _CHUNK)])
            return carry

        lax.fori_loop(0, nchunk, chunk, 0)

    return k(ids, emb)


def _tc_argmin(baseh, z2, cbtp, c2p):
    nk = _K // _KT

    def body(base_ref, z2_ref, cb_ref, c2_ref, out_ref):
        ba = base_ref[...]
        zz = z2_ref[...]
        bestv = None
        bestk = None
        for kk in range(nk):
            cb = cb_ref[:, kk * _KT:(kk + 1) * _KT]
            d2 = jnp.dot(ba, cb, preferred_element_type=jnp.float32)
            d = (zz + d2) + c2_ref[:, kk * _KT:(kk + 1) * _KT]
            if kk == 0:
                bestv = d
                bestk = jnp.zeros((_TT, _KT), jnp.int32)
            else:
                upd = d < bestv
                bestv = jnp.where(upd, d, bestv)
                bestk = jnp.where(upd, kk, bestk)
        m = jnp.min(bestv, axis=1, keepdims=True)
        lane = lax.broadcasted_iota(jnp.int32, (_TT, _KT), 1)
        idxfull = lane * nk + bestk
        elig = jnp.where(bestv == m, idxfull, jnp.int32(2**30))
        out_ref[...] = jnp.min(elig, axis=1, keepdims=True)

    return pl.pallas_call(
        body,
        grid=(_T // _TT,),
        in_specs=[
            pl.BlockSpec((_TT, _D), lambda t: (t, 0)),
            pl.BlockSpec((_TT, 1), lambda t: (t, 0)),
            pl.BlockSpec((_D, _K), lambda t: (0, 0)),
            pl.BlockSpec((1, _K), lambda t: (0, 0)),
        ],
        out_specs=pl.BlockSpec((_TT, 1), lambda t: (t, 0)),
        out_shape=jax.ShapeDtypeStruct((_T, 1), jnp.int32),
    )(baseh, z2, cbtp, c2p)


def _mix_gather(qidx, off, codebook):
    tok_per_chunk = _CHUNK // _KMIX
    nchunk = _TPW // tok_per_chunk
    opw = _TPW * _KMIX

    @functools.partial(
        pl.kernel,
        out_type=jax.ShapeDtypeStruct((_T, _D), jnp.float32),
        mesh=_sc_mesh(),
        compiler_params=pltpu.CompilerParams(
            use_tc_tiling_on_sc=False, needs_layout_passes=False),
        scratch_types=[
            pltpu.VMEM((_T,), jnp.int32),
            pltpu.VMEM((opw,), jnp.int32),
            pltpu.VMEM((_CHUNK,), jnp.int32),
            pltpu.VMEM((_CHUNK, _D), jnp.float32),
            pltpu.VMEM((_TPW, _D), jnp.float32),
            pltpu.SemaphoreType.DMA,
        ],
    )
    def k(qidx_hbm, off_hbm, cb_hbm, out_hbm,
          qidx_v, off_v, codes_v, rows_v, out_v, sem):
        wid = lax.axis_index("s") * _NC + lax.axis_index("c")
        pltpu.sync_copy(qidx_hbm, qidx_v)
        pltpu.sync_copy(off_hbm.at[pl.ds(wid * opw, opw)], off_v)

        def chunk(c, carry):
            cbase = c * _CHUNK
            for g in range(_CHUNK // 16):
                off16 = off_v[pl.ds(cbase + g * 16, 16)]
                codes_v[pl.ds(g * 16, 16)] = plsc.load_gather(qidx_v, [off16])
            pltpu.async_copy(cb_hbm.at[codes_v], rows_v, sem).wait()
            for i in range(tok_per_chunk):
                for h in range(_D // 16):
                    acc = rows_v[i * _KMIX, pl.ds(h * 16, 16)]
                    for j in range(1, _KMIX):
                        acc = acc + rows_v[i * _KMIX + j, pl.ds(h * 16, 16)]
                    out_v[c * tok_per_chunk + i, pl.ds(h * 16, 16)] = (
                        acc * (1.0 / _KMIX))
            return carry

        lax.fori_loop(0, nchunk, chunk, 0)
        pltpu.sync_copy(out_v, out_hbm.at[pl.ds(wid * _TPW, _TPW)])

    return k(qidx, off, codebook)


def kernel(input_ids, attention_mask, token_embedding, codebook):
    ids = input_ids.reshape(-1)
    maskf = attention_mask.reshape(-1, 1).astype(jnp.float32)
    base = _embed_gather(ids, token_embedding)
    bm = base * maskf
    z2 = jnp.sum(bm * bm, axis=-1, keepdims=True)
    baseh = bm.astype(jnp.bfloat16)
    cbt2 = (-2.0 * codebook.T).astype(jnp.bfloat16)
    c2 = jnp.sum(codebook * codebook, axis=-1)[None]

    def _perm(a):
        return a.reshape(a.shape[0], _KT, _K // _KT).transpose(0, 2, 1) \
                .reshape(a.shape[0], _K)

    qidx = _tc_argmin(baseh, z2, _perm(cbt2), _perm(c2)).reshape(-1)
    mix = jax.random.randint(jax.random.key(1), (_B, _S, _KMIX), 0, _B)
    off = (mix * _S
           + jnp.arange(_S, dtype=jnp.int32)[None, :, None]).reshape(-1)
    mixed = _mix_gather(qidx, off.astype(jnp.int32), codebook)
    return mixed.reshape(_B, _S, _D)
